# baseline (speedup 1.0000x reference)
import functools

import jax
import jax.numpy as jnp
from jax import lax
from jax.experimental import pallas as pl
from jax.experimental.pallas import tpu as pltpu
from jax.experimental.pallas import tpu_sc as plsc

B, S, D = 2, 4096, 64
BS = 64
NB = S // BS
NBLK = B * NB * NB
M_EMB = 2 * S - 1
M_PAD = 2 * S
NROWS = B * S

NWORKERS = 32
NTASK = B * (NB // 2) * BS
TPW = NTASK // NWORKERS


def _mm_body(q_ref, e_ref, o_ref):
    q = q_ref[0].astype(jnp.bfloat16)
    e = e_ref[0].astype(jnp.bfloat16)
    o_ref[0] = lax.dot_general(q, e, (((1,), (1,)), ((), ())),
                               preferred_element_type=jnp.float32)


def _scores(q, emb_pad, interpret=False):
    BM, BN = 1024, 2048
    return pl.pallas_call(
        _mm_body,
        grid=(B, S // BM, M_PAD // BN),
        in_specs=[pl.BlockSpec((1, BM, D), lambda b, i, j: (b, i, 0)),
                  pl.BlockSpec((1, BN, D), lambda b, i, j: (b, j, 0))],
        out_specs=pl.BlockSpec((1, BM, BN), lambda b, i, j: (b, i, j)),
        out_shape=jax.ShapeDtypeStruct((B, S, M_PAD), jnp.float32),
        interpret=interpret,
    )(q, emb_pad)


def _gather_body(scores_hbm, info_hbm, out_hbm,
                 rowa0, rowa1, rowb0, rowb1, idx0, idx1, o0, o1,
                 sr0, sr1, si0, si1, so0, so1):
    wid = lax.axis_index("s") * 2 + lax.axis_index("c")
    bufs = ((rowa0, rowb0, idx0, o0, sr0, si0, so0),
            (rowa1, rowb1, idx1, o1, sr1, si1, so1))

    def params(t):
        g = wid * TPW + t
        b = g // 2048
        gg = g % 2048
        rp = gg // BS
        i = gg % BS
        row_a = b * S + rp * 128 + i
        nt = b * 32 + rp
        return row_a, i, nt

    def start_in(t, buf):
        row_a_v, row_b_v, idx_v, _, sr, si, _ = buf
        row_a, i, nt = params(t)
        pltpu.async_copy(scores_hbm.at[row_a], row_a_v, sr)
        pltpu.async_copy(scores_hbm.at[row_a + BS], row_b_v, sr)
        pltpu.async_copy(info_hbm.at[i, :, pl.ds(nt * 1024, 1024)], idx_v, si)

    def wait_in(buf):
        row_a_v, row_b_v, idx_v, _, sr, si, _ = buf
        pltpu.make_async_copy(scores_hbm.at[0], row_a_v, sr).wait()
        pltpu.make_async_copy(scores_hbm.at[0], row_b_v, sr).wait()
        pltpu.make_async_copy(
            info_hbm.at[0, :, pl.ds(0, 1024)], idx_v, si).wait()

    def wait_out(buf):
        out_v, so = buf[3], buf[6]
        pltpu.make_async_copy(
            out_v, out_hbm.at[0, :, pl.ds(0, 1024)], so).wait()

    def compute(t, buf):
        row_a_v, row_b_v, idx_v, out_v, _, _, so = buf

        def col(jt, c2):
            for js in range(8):
                base = js * 128
                for c0 in range(0, BS, 16):
                    idx = idx_v[jt, pl.ds(base + c0, 16)]
                    out_v[jt, pl.ds(base + c0, 16)] = plsc.load_gather(
                        row_a_v, [idx])
                for c0 in range(0, BS, 16):
                    idx = idx_v[jt, pl.ds(base + BS + c0, 16)]
                    out_v[jt, pl.ds(base + BS + c0, 16)] = plsc.load_gather(
                        row_b_v, [idx])
            return c2

        lax.fori_loop(0, 8, col, 0)
        _, i, nt = params(t)
        pltpu.async_copy(out_v, out_hbm.at[i, :, pl.ds(nt * 1024, 1024)], so)

    start_in(0, bufs[0])
    start_in(1, bufs[1])

    def outer(tt, carry):
        t0 = 2 * tt
        for p in range(2):
            buf = bufs[p]
            wait_in(buf)

            @pl.when(tt > 0)
            def _():
                wait_out(buf)

            compute(t0 + p, buf)

            @pl.when(t0 + p + 2 < TPW)
            def _():
                start_in(t0 + p + 2, buf)
        return carry

    lax.fori_loop(0, TPW // 2, outer, 0)
    wait_out(bufs[0])
    wait_out(bufs[1])


def _gather(scores, info3):
    mesh = plsc.VectorSubcoreMesh(core_axis_name="c", subcore_axis_name="s")
    f = pl.kernel(
        _gather_body,
        mesh=mesh,
        out_type=jax.ShapeDtypeStruct((BS, 8, NBLK * 8), jnp.float32),
        scratch_types=[
            pltpu.VMEM((M_PAD,), jnp.float32),
            pltpu.VMEM((M_PAD,), jnp.float32),
            pltpu.VMEM((M_PAD,), jnp.float32),
            pltpu.VMEM((M_PAD,), jnp.float32),
            pltpu.VMEM((8, 1024), jnp.int32),
            pltpu.VMEM((8, 1024), jnp.int32),
            pltpu.VMEM((8, 1024), jnp.float32),
            pltpu.VMEM((8, 1024), jnp.float32),
            pltpu.SemaphoreType.DMA,
            pltpu.SemaphoreType.DMA,
            pltpu.SemaphoreType.DMA,
            pltpu.SemaphoreType.DMA,
            pltpu.SemaphoreType.DMA,
            pltpu.SemaphoreType.DMA,
        ],
        compiler_params=pltpu.CompilerParams(needs_layout_passes=False),
    )
    return f(scores, info3)


def kernel(q, emb, info, sparsity_layout):
    del sparsity_layout
    emb_pad = jnp.concatenate([emb, jnp.zeros((B, 1, D), emb.dtype)], axis=1)
    scores = _scores(q, emb_pad).reshape(NROWS, M_PAD)
    info3 = jnp.transpose(
        info.reshape(NBLK // 128, 128, BS, 8, 8),
        (2, 3, 0, 4, 1)).reshape(BS, 8, NBLK * 8)
    out3 = _gather(scores, info3)
    return jnp.transpose(
        out3.reshape(BS, 8, NBLK // 128, 8, 128),
        (2, 4, 0, 1, 3)).reshape(NBLK, BS, BS)


# speedup vs baseline: 1.4455x; 1.4455x over previous
import functools

import jax
import jax.numpy as jnp
from jax import lax
from jax.experimental import pallas as pl
from jax.experimental.pallas import tpu as pltpu
from jax.experimental.pallas import tpu_sc as plsc

B, S, D = 2, 4096, 64
BS = 64
NB = S // BS
NBLK = B * NB * NB
M_EMB = 2 * S - 1
M_PAD = 2 * S
NROWS = B * S

NWORKERS = 32
NTASK = B * (NB // 2) * BS
TPW = NTASK // NWORKERS


def _mm_body(q_ref, e_ref, o_ref):
    q = q_ref[0].astype(jnp.bfloat16)
    e = e_ref[0].astype(jnp.bfloat16)
    o_ref[0] = lax.dot_general(q, e, (((1,), (1,)), ((), ())),
                               preferred_element_type=jnp.float32)


def _scores(q, emb_pad, interpret=False):
    BM, BN = 1024, 2048
    return pl.pallas_call(
        _mm_body,
        grid=(B, S // BM, M_PAD // BN),
        in_specs=[pl.BlockSpec((1, BM, D), lambda b, i, j: (b, i, 0)),
                  pl.BlockSpec((1, BN, D), lambda b, i, j: (b, j, 0))],
        out_specs=pl.BlockSpec((1, BM, BN), lambda b, i, j: (b, i, j)),
        out_shape=jax.ShapeDtypeStruct((B, S, M_PAD), jnp.float32),
        interpret=interpret,
    )(q, emb_pad)


def _gather_body(scores_hbm, info_hbm, out_hbm,
                 rowa0, rowa1, rowb0, rowb1, idx0, idx1, o0, o1,
                 sr0, sr1, si0, si1, so0, so1):
    wid = lax.axis_index("s") * 2 + lax.axis_index("c")
    bufs = ((rowa0, rowb0, idx0, o0, sr0, si0, so0),
            (rowa1, rowb1, idx1, o1, sr1, si1, so1))

    def params(t):
        g = wid * TPW + t
        b = g // 2048
        gg = g % 2048
        rp = gg // BS
        i = gg % BS
        row_a = b * S + rp * 128 + i
        nt = b * 32 + rp
        return row_a, i, nt

    def start_in(t, buf):
        row_a_v, row_b_v, idx_v, _, sr, si, _ = buf
        row_a, i, nt = params(t)
        pltpu.async_copy(scores_hbm.at[row_a], row_a_v, sr)
        pltpu.async_copy(scores_hbm.at[row_a + BS], row_b_v, sr)
        pltpu.async_copy(
            info_hbm.at[i, :, :, pl.ds(nt * 128, 128)], idx_v, si)

    def wait_in(buf):
        row_a_v, row_b_v, idx_v, _, sr, si, _ = buf
        pltpu.make_async_copy(scores_hbm.at[0], row_a_v, sr).wait()
        pltpu.make_async_copy(scores_hbm.at[0], row_b_v, sr).wait()
        pltpu.make_async_copy(
            info_hbm.at[0, :, :, pl.ds(0, 128)], idx_v, si).wait()

    def wait_out(buf):
        out_v, so = buf[3], buf[6]
        pltpu.make_async_copy(
            out_v, out_hbm.at[0, :, :, pl.ds(0, 128)], so).wait()

    def compute(t, buf):
        row_a_v, row_b_v, idx_v, out_v, _, _, so = buf

        def col(jt, c2):
            for js in range(8):
                for c0 in range(0, BS, 16):
                    idx = idx_v[jt, js, pl.ds(c0, 16)]
                    out_v[jt, js, pl.ds(c0, 16)] = plsc.load_gather(
                        row_a_v, [idx])
                for c0 in range(0, BS, 16):
                    idx = idx_v[jt, js, pl.ds(BS + c0, 16)]
                    out_v[jt, js, pl.ds(BS + c0, 16)] = plsc.load_gather(
                        row_b_v, [idx])
            return c2

        lax.fori_loop(0, 8, col, 0)
        _, i, nt = params(t)
        pltpu.async_copy(
            out_v, out_hbm.at[i, :, :, pl.ds(nt * 128, 128)], so)

    start_in(0, bufs[0])
    start_in(1, bufs[1])

    def outer(tt, carry):
        t0 = 2 * tt
        for p in range(2):
            buf = bufs[p]
            wait_in(buf)

            @pl.when(tt > 0)
            def _():
                wait_out(buf)

            compute(t0 + p, buf)

            @pl.when(t0 + p + 2 < TPW)
            def _():
                start_in(t0 + p + 2, buf)
        return carry

    lax.fori_loop(0, TPW // 2, outer, 0)
    wait_out(bufs[0])
    wait_out(bufs[1])


def _gather(scores, info3):
    mesh = plsc.VectorSubcoreMesh(core_axis_name="c", subcore_axis_name="s")
    f = pl.kernel(
        _gather_body,
        mesh=mesh,
        out_type=jax.ShapeDtypeStruct((BS, 8, 8, NBLK), jnp.float32),
        scratch_types=[
            pltpu.VMEM((M_PAD,), jnp.float32),
            pltpu.VMEM((M_PAD,), jnp.float32),
            pltpu.VMEM((M_PAD,), jnp.float32),
            pltpu.VMEM((M_PAD,), jnp.float32),
            pltpu.VMEM((8, 8, 128), jnp.int32),
            pltpu.VMEM((8, 8, 128), jnp.int32),
            pltpu.VMEM((8, 8, 128), jnp.float32),
            pltpu.VMEM((8, 8, 128), jnp.float32),
            pltpu.SemaphoreType.DMA,
            pltpu.SemaphoreType.DMA,
            pltpu.SemaphoreType.DMA,
            pltpu.SemaphoreType.DMA,
            pltpu.SemaphoreType.DMA,
            pltpu.SemaphoreType.DMA,
        ],
        compiler_params=pltpu.CompilerParams(needs_layout_passes=False),
    )
    return f(scores, info3)


def kernel(q, emb, info, sparsity_layout):
    del sparsity_layout
    emb_pad = jnp.concatenate([emb, jnp.zeros((B, 1, D), emb.dtype)], axis=1)
    scores = _scores(q, emb_pad).reshape(NROWS, M_PAD)
    info4 = jnp.transpose(
        info.reshape(NBLK // 128, 128, BS, 8, 8),
        (2, 3, 4, 0, 1)).reshape(BS, 8, 8, NBLK)
    out4 = _gather(scores, info4)
    return jnp.transpose(
        out4.reshape(BS, 8, 8, NBLK // 128, 128),
        (3, 4, 0, 1, 2)).reshape(NBLK, BS, BS)
